# baseline (speedup 1.0000x reference)
import functools

import jax
import jax.numpy as jnp
from jax import lax
from jax.experimental import pallas as pl
from jax.experimental.pallas import tpu as pltpu
from jax.experimental.pallas import tpu_sc as plsc

_NUM_CORES = 2
_NUM_SUBCORES = 16
_NUM_WORKERS = _NUM_CORES * _NUM_SUBCORES
_LANES = 16


def _gather_sc(i, Xb, B, D):
    R = Xb.shape[1] // D
    b_per_w = B // _NUM_WORKERS
    n_dma = b_per_w // 128
    n_grp = b_per_w // _LANES
    mesh = plsc.VectorSubcoreMesh(core_axis_name="c", subcore_axis_name="s")

    @functools.partial(
        pl.kernel,
        mesh=mesh,
        out_type=jax.ShapeDtypeStruct((B * D,), Xb.dtype),
        compiler_params=pltpu.CompilerParams(needs_layout_passes=False),
        scratch_types=[
            pltpu.VMEM((b_per_w,), jnp.int32),
            pltpu.VMEM((n_dma, 128), jnp.int32),
            pltpu.VMEM((b_per_w, D * R), Xb.dtype),
            pltpu.VMEM((b_per_w * D,), Xb.dtype),
            pltpu.SemaphoreType.DMA,
        ],
    )
    def k(table_hbm, idx_hbm, out_hbm, idx_v, blk_v, rows_v, out_v, sem):
        wid = lax.axis_index("s") * _NUM_CORES + lax.axis_index("c")
        base = wid * b_per_w
        pltpu.sync_copy(idx_hbm.at[pl.ds(base, b_per_w)], idx_v)

        @pl.loop(0, n_dma)
        def _(c):
            for w in range(128 // _LANES):
                v = idx_v[pl.ds(c * 128 + w * _LANES, _LANES)]
                blk_v[c, pl.ds(w * _LANES, _LANES)] = v >> 3

        copies = [
            pltpu.async_copy(
                table_hbm.at[blk_v.at[c]],
                rows_v.at[pl.ds(c * 128, 128)],
                sem,
            )
            for c in range(n_dma)
        ]
        for c in copies:
            c.wait()

        lane = lax.iota(jnp.int32, _LANES)

        @pl.loop(0, n_grp)
        def _(u):
            v = idx_v[pl.ds(u * _LANES, _LANES)]
            colb = (v & 7) * D
            r = u * _LANES + lane
            o0 = r * D
            for kk in range(D):
                val = plsc.load_gather(rows_v, [r, colb + kk])
                plsc.store_scatter(out_v, [o0 + kk], val)

        pltpu.sync_copy(out_v, out_hbm.at[pl.ds(base * D, b_per_w * D)])

    return k(Xb, i)


def kernel(i, X):
    B = i.shape[0]
    V, D = X.shape
    Xb = X.reshape(V * D // 128, 128)
    out = _gather_sc(i.astype(jnp.int32), Xb, B, D)
    return out.reshape(B, D)
